# speedup vs baseline: 2.4174x; 2.4174x over previous
import functools
import math

import jax
import jax.numpy as jnp
from jax import lax
from jax.experimental import pallas as pl
from jax.experimental.pallas import tpu as pltpu
from jax.experimental.pallas import tpu_sc as plsc

D_MODEL = 128
SCALE = math.sqrt(128.0)
NUM_CORES = 2
NUM_SUBCORES = 16
NUM_WORKERS = NUM_CORES * NUM_SUBCORES
CHUNK = 128


@functools.partial(jax.jit, static_argnames=("nsteps",))
def _embed_sc(idx, table, nsteps):
    rows_per_worker = nsteps * CHUNK
    total_rows = NUM_WORKERS * rows_per_worker

    @functools.partial(
        pl.kernel,
        out_type=jax.ShapeDtypeStruct((total_rows, D_MODEL), jnp.float32),
        mesh=plsc.VectorSubcoreMesh(core_axis_name="c", subcore_axis_name="s"),
        scratch_types=[
            pltpu.VMEM((nsteps, CHUNK), jnp.int32),
            pltpu.VMEM((CHUNK, D_MODEL), jnp.float32),
            pltpu.SemaphoreType.DMA,
        ],
    )
    def k(idx_hbm, table_hbm, out_hbm, idx_v, buf, sem):
        wid = lax.axis_index("s") * NUM_CORES + lax.axis_index("c")
        pltpu.sync_copy(idx_hbm.at[wid], idx_v)
        base = wid * rows_per_worker

        def step(g, carry):
            pltpu.async_copy(table_hbm.at[idx_v.at[g]], buf, sem).wait()

            def scale_row(r, c2):
                for j in range(D_MODEL // 16):
                    sl = pl.ds(j * 16, 16)
                    buf[r, sl] = buf[r, sl] * SCALE
                return c2

            lax.fori_loop(0, CHUNK, scale_row, 0)
            pltpu.sync_copy(buf, out_hbm.at[pl.ds(base + g * CHUNK, CHUNK)])
            return carry

        lax.fori_loop(0, nsteps, step, 0)

    return k(idx, table)


def kernel(x, word_emb):
    total = x.size
    rows_per_worker = total // NUM_WORKERS
    nsteps = rows_per_worker // CHUNK
    idx = x.reshape(NUM_WORKERS, nsteps, CHUNK).astype(jnp.int32)
    out = _embed_sc(idx, word_emb, nsteps)
    return out.reshape(x.shape + (D_MODEL,))


# speedup vs baseline: 2.8535x; 1.1804x over previous
import functools
import math

import jax
import jax.numpy as jnp
from jax import lax
from jax.experimental import pallas as pl
from jax.experimental.pallas import tpu as pltpu
from jax.experimental.pallas import tpu_sc as plsc

D_MODEL = 128
SCALE = math.sqrt(128.0)
NUM_CORES = 2
NUM_SUBCORES = 16
NUM_WORKERS = NUM_CORES * NUM_SUBCORES
CHUNK = 128


@functools.partial(jax.jit, static_argnames=("nsteps",))
def _embed_sc(idx, table, nsteps):
    rows_per_worker = nsteps * CHUNK
    total_rows = NUM_WORKERS * rows_per_worker

    npairs = nsteps // 2

    @functools.partial(
        pl.kernel,
        out_type=jax.ShapeDtypeStruct((total_rows, D_MODEL), jnp.float32),
        mesh=plsc.VectorSubcoreMesh(core_axis_name="c", subcore_axis_name="s"),
        scratch_types=[
            pltpu.VMEM((nsteps, CHUNK), jnp.int32),
            pltpu.VMEM((CHUNK, D_MODEL), jnp.float32),
            pltpu.VMEM((CHUNK, D_MODEL), jnp.float32),
            pltpu.SemaphoreType.DMA,
            pltpu.SemaphoreType.DMA,
            pltpu.SemaphoreType.DMA,
            pltpu.SemaphoreType.DMA,
        ],
    )
    def k(idx_hbm, table_hbm, out_hbm, idx_v, buf0, buf1, g0sem, g1sem,
          o0sem, o1sem):
        wid = lax.axis_index("s") * NUM_CORES + lax.axis_index("c")
        pltpu.sync_copy(idx_hbm.at[wid], idx_v)
        base = wid * rows_per_worker

        def scale(buf):
            def quad(q, c2):
                r0 = q * 4
                for r in range(4):
                    for j in range(D_MODEL // 16):
                        sl = pl.ds(j * 16, 16)
                        buf[r0 + r, sl] = buf[r0 + r, sl] * SCALE
                return c2

            lax.fori_loop(0, CHUNK // 4, quad, 0)

        def gather_start(g, buf, sem):
            pltpu.async_copy(table_hbm.at[idx_v.at[g]], buf, sem)

        def gather_wait(g, buf, sem):
            pltpu.make_async_copy(table_hbm.at[idx_v.at[g]], buf, sem).wait()

        def put_start(g, buf, sem):
            pltpu.async_copy(
                buf, out_hbm.at[pl.ds(base + g * CHUNK, CHUNK)], sem)

        def put_wait(g, buf, sem):
            pltpu.make_async_copy(
                buf, out_hbm.at[pl.ds(base + g * CHUNK, CHUNK)], sem).wait()

        gather_start(0, buf0, g0sem)
        gather_start(1, buf1, g1sem)

        def pair(h, carry):
            g0 = h * 2
            g1 = g0 + 1
            gather_wait(g0, buf0, g0sem)
            scale(buf0)
            put_start(g0, buf0, o0sem)
            gather_wait(g1, buf1, g1sem)
            scale(buf1)
            put_start(g1, buf1, o1sem)

            @pl.when(h + 1 < npairs)
            def _():
                put_wait(g0, buf0, o0sem)
                gather_start(g0 + 2, buf0, g0sem)
                put_wait(g1, buf1, o1sem)
                gather_start(g1 + 2, buf1, g1sem)

            return carry

        lax.fori_loop(0, npairs, pair, 0)
        put_wait(nsteps - 2, buf0, o0sem)
        put_wait(nsteps - 1, buf1, o1sem)

    return k(idx, table)


def kernel(x, word_emb):
    total = x.size
    rows_per_worker = total // NUM_WORKERS
    nsteps = rows_per_worker // CHUNK
    idx = x.reshape(NUM_WORKERS, nsteps, CHUNK).astype(jnp.int32)
    out = _embed_sc(idx, word_emb, nsteps)
    return out.reshape(x.shape + (D_MODEL,))


# speedup vs baseline: 4.1743x; 1.4628x over previous
import functools
import math

import jax
import jax.numpy as jnp
from jax import lax
from jax.experimental import pallas as pl
from jax.experimental.pallas import tpu as pltpu
from jax.experimental.pallas import tpu_sc as plsc

D_MODEL = 128
SCALE = math.sqrt(128.0)
NUM_CORES = 2
NUM_SUBCORES = 16
NUM_WORKERS = NUM_CORES * NUM_SUBCORES
SEQ = 50


@functools.partial(jax.jit, static_argnames=("batches",))
def _embed_sc(idx, table, batches):
    b_per_w = batches // NUM_WORKERS
    npairs = b_per_w // 2

    @functools.partial(
        pl.kernel,
        out_type=jax.ShapeDtypeStruct((batches, SEQ, D_MODEL), jnp.float32),
        mesh=plsc.VectorSubcoreMesh(core_axis_name="c", subcore_axis_name="s"),
        compiler_params=pltpu.CompilerParams(use_tc_tiling_on_sc=True),
        scratch_types=[
            pltpu.VMEM((b_per_w, SEQ), jnp.int32),
            pltpu.VMEM((SEQ, D_MODEL), jnp.float32),
            pltpu.VMEM((SEQ, D_MODEL), jnp.float32),
            pltpu.SemaphoreType.DMA,
            pltpu.SemaphoreType.DMA,
            pltpu.SemaphoreType.DMA,
            pltpu.SemaphoreType.DMA,
        ],
    )
    def k(idx_hbm, table_hbm, out_hbm, idx_v, buf0, buf1, g0sem, g1sem,
          o0sem, o1sem):
        wid = lax.axis_index("s") * NUM_CORES + lax.axis_index("c")
        pltpu.sync_copy(idx_hbm.at[wid], idx_v)
        base = wid * b_per_w

        def scale(buf):
            def body(q, c2):
                r0 = q * 5
                for r in range(5):
                    for j in range(D_MODEL // 16):
                        sl = pl.ds(j * 16, 16)
                        buf[r0 + r, sl] = buf[r0 + r, sl] * SCALE
                return c2

            lax.fori_loop(0, SEQ // 5, body, 0)

        def gather_start(g, buf, sem):
            pltpu.async_copy(table_hbm.at[idx_v.at[g]], buf, sem)

        def gather_wait(g, buf, sem):
            pltpu.make_async_copy(table_hbm.at[idx_v.at[g]], buf, sem).wait()

        def put_start(g, buf, sem):
            pltpu.async_copy(buf, out_hbm.at[base + g], sem)

        def put_wait(g, buf, sem):
            pltpu.make_async_copy(buf, out_hbm.at[base + g], sem).wait()

        gather_start(0, buf0, g0sem)
        gather_start(1, buf1, g1sem)

        def pair(h, carry):
            g0 = h * 2
            g1 = g0 + 1
            gather_wait(g0, buf0, g0sem)
            scale(buf0)
            put_start(g0, buf0, o0sem)
            gather_wait(g1, buf1, g1sem)
            scale(buf1)
            put_start(g1, buf1, o1sem)

            @pl.when(h + 1 < npairs)
            def _():
                put_wait(g0, buf0, o0sem)
                gather_start(g0 + 2, buf0, g0sem)
                put_wait(g1, buf1, o1sem)
                gather_start(g1 + 2, buf1, g1sem)

            return carry

        lax.fori_loop(0, npairs, pair, 0)
        put_wait(b_per_w - 2, buf0, o0sem)
        put_wait(b_per_w - 1, buf1, o1sem)

    return k(idx, table)


def kernel(x, word_emb):
    batches = x.shape[0]
    idx = x.reshape(NUM_WORKERS, batches // NUM_WORKERS, SEQ).astype(jnp.int32)
    return _embed_sc(idx, word_emb, batches)
